# baseline (speedup 1.0000x reference)
import functools

import jax
import jax.numpy as jnp
from jax import lax
from jax.experimental import pallas as pl
from jax.experimental.pallas import tpu as pltpu
from jax.experimental.pallas import tpu_sc as plsc

_N = 10000
_H = 128
_E = 320000
_B0 = 160
_B1 = 80
_BG = 80
_BM = 2000
_NP = 10112
_RPT = 632

_f32 = jnp.float32


def _ln_relu(x, g, b):
    mu = jnp.mean(x, axis=-1, keepdims=True)
    var = jnp.mean((x - mu) ** 2, axis=-1, keepdims=True)
    return jnp.maximum((x - mu) / jnp.sqrt(var + 1e-5) * g + b, 0.0)


def _tc1_body(xv, xa, wv, wa, hv, ha):
    hv[...] = jnp.dot(xv[...], wv[...], preferred_element_type=_f32)
    ha[...] = jnp.dot(xa[...], wa[...], preferred_element_type=_f32)


def _tc1(xv, xa, wv, wa):
    return pl.pallas_call(
        _tc1_body,
        out_shape=[jax.ShapeDtypeStruct((_N, _H), _f32),
                   jax.ShapeDtypeStruct((_N, _H), _f32)],
    )(xv, xa, wv, wa)


def _tc2_body(vraw, araw, bv0, ba0, g0v, b0v, g0a, b0a,
              wsh, wgs, wgd, a_s, a_d,
              hv1, ha1, hs, s1, s2, shift):
    v = _ln_relu(vraw[...] + bv0[...][None, :], g0v[...][None, :], b0v[...][None, :])
    a = _ln_relu(araw[...] + ba0[...][None, :], g0a[...][None, :], b0a[...][None, :])
    hv1[...] = jnp.dot(v, wsh[...], preferred_element_type=_f32)
    ha1[...] = jnp.dot(a, wsh[...], preferred_element_type=_f32)
    hsv = jnp.dot(v, wgs[...], preferred_element_type=_f32)
    hs[...] = hsv
    s1v = jnp.sum(hsv * a_s[...][None, :], axis=1, keepdims=True)
    w2 = jnp.dot(wgd[...], a_d[...][:, None], preferred_element_type=_f32)
    s2v = jnp.dot(a, w2, preferred_element_type=_f32)
    s1[...] = s1v
    s2[...] = s2v
    sh = jnp.maximum(jnp.max(s1v) + jnp.max(s2v), 0.0)
    shift[...] = jnp.full((8, 128), sh, _f32)


def _tc2(vraw, araw, bv0, ba0, g0v, b0v, g0a, b0a, wsh, wgs, wgd, a_s, a_d):
    return pl.pallas_call(
        _tc2_body,
        out_shape=[jax.ShapeDtypeStruct((_N, _H), _f32),
                   jax.ShapeDtypeStruct((_N, _H), _f32),
                   jax.ShapeDtypeStruct((_N, _H), _f32),
                   jax.ShapeDtypeStruct((_N, 1), _f32),
                   jax.ShapeDtypeStruct((_N, 1), _f32),
                   jax.ShapeDtypeStruct((8, 128), _f32)],
    )(vraw, araw, bv0, ba0, g0v, b0v, g0a, b0a, wsh, wgs, wgd, a_s, a_d)


def _tc3a_body(v1p0, v1p1, a1p0, a1p1, np0, np1, dp0, dp1,
               bsh, bgat, g1v, b1v, g1a, b1a, attw_a, attw_v,
               v1, a1, sv, sa):
    v1raw = v1p0[...] + v1p1[...]
    a1raw = a1p0[...] + a1p1[...]
    num = np0[...] + np1[...]
    den = dp0[...] + dp1[...]
    gat = num / (den + 1e-16)
    v1b = _ln_relu(v1raw + bsh[...][None, :], g1v[...][None, :], b1v[...][None, :])
    a1b = _ln_relu(a1raw + gat + (bsh[...] + bgat[...])[None, :],
                   g1a[...][None, :], b1a[...][None, :])
    v1[...] = v1b
    a1[...] = a1b
    sv[...] = jnp.sum(v1b * attw_v[...][None, :], axis=1, keepdims=True)
    sa[...] = jnp.sum(a1b * attw_a[...][None, :], axis=1, keepdims=True)


_BLK = 632


def _tc3a(v1p, a1p, nump, denp, bsh, bgat, g1v, b1v, g1a, b1a, attw_a, attw_v):
    nblk = _NP // _BLK
    half0 = pl.BlockSpec((_BLK, _H), lambda i: (i, 0))
    half1 = pl.BlockSpec((_BLK, _H), lambda i: (i + nblk, 0))
    dhalf0 = pl.BlockSpec((_BLK, 1), lambda i: (i, 0))
    dhalf1 = pl.BlockSpec((_BLK, 1), lambda i: (i + nblk, 0))
    vec = pl.BlockSpec((_H,), lambda i: (0,))
    return pl.pallas_call(
        _tc3a_body,
        grid=(nblk,),
        in_specs=[half0, half1, half0, half1, half0, half1, dhalf0, dhalf1,
                  vec, vec, vec, vec, vec, vec, vec, vec],
        out_specs=[pl.BlockSpec((_BLK, _H), lambda i: (i, 0)),
                   pl.BlockSpec((_BLK, _H), lambda i: (i, 0)),
                   pl.BlockSpec((_BLK, 1), lambda i: (i, 0)),
                   pl.BlockSpec((_BLK, 1), lambda i: (i, 0))],
        out_shape=[jax.ShapeDtypeStruct((_NP, _H), _f32),
                   jax.ShapeDtypeStruct((_NP, _H), _f32),
                   jax.ShapeDtypeStruct((_NP, 1), _f32),
                   jax.ShapeDtypeStruct((_NP, 1), _f32)],
    )(v1p, v1p, a1p, a1p, nump, nump, denp, denp,
      bsh, bgat, g1v, b1v, g1a, b1a, attw_a, attw_v)


def _tc3b_body(v1, a1, sv, sa, lin_w, lin_b, out):
    rowid = lax.broadcasted_iota(jnp.int32, (_NP, 1), 0)
    mask = rowid < _N

    def readout(x, s):
        m = jnp.max(jnp.where(mask, s, -jnp.inf))
        ex = jnp.where(mask, jnp.exp(s - m), 0.0)
        gate = ex / jnp.sum(ex)
        return jnp.sum(gate * x, axis=0, keepdims=True)

    ga_a = readout(a1[...], sa[...])
    ga_v = readout(v1[...], sv[...])
    both = jnp.concatenate([ga_a, ga_v], axis=0)
    out[...] = jnp.dot(both, lin_w[...], preferred_element_type=_f32) \
        + lin_b[...][None, :]


def _tc3b(v1, a1, sv, sa, lin_w, lin_b):
    return pl.pallas_call(
        _tc3b_body,
        out_shape=jax.ShapeDtypeStruct((2, _H), _f32),
    )(v1, a1, sv, sa, lin_w, lin_b)


_MESH = plsc.VectorSubcoreMesh(core_axis_name="c", subcore_axis_name="s")


def _zero_acc(z2d, acc, t):
    pltpu.sync_copy(z2d, acc.at[pl.ds(t * _RPT, _RPT)])


def _gcn_stage(src, dst, tab, s, base, bsz):
    isx, idx, rws, gs, ss = s
    pltpu.sync_copy(src.at[pl.ds(base, bsz)], isx)
    pltpu.sync_copy(dst.at[pl.ds(base, bsz)], idx)
    pltpu.async_copy(tab.at[isx], rws, gs)


def _gcn_prime(src, dst, tab, bufs, base0, nbatch, bsz):
    _gcn_stage(src, dst, tab, bufs[0], base0, bsz)
    if nbatch > 1:
        _gcn_stage(src, dst, tab, bufs[1], base0 + bsz, bsz)


def _gcn_pipe(src, dst, tab, acc, bufs, base0, nbatch, bsz):
    s0, s1 = bufs

    def gwait(s):
        pltpu.make_async_copy(tab.at[s[0]], s[2], s[3]).wait()

    def sstart(s):
        pltpu.async_copy(s[2], acc.at[s[1]], s[4], add=True)

    def swait(s):
        pltpu.make_async_copy(s[2], acc.at[s[1]], s[4]).wait()

    @pl.loop(0, nbatch // 2)
    def _(j):
        b = 2 * j
        gwait(s0)
        sstart(s0)
        gwait(s1)
        sstart(s1)
        swait(s0)

        @pl.when(b + 2 < nbatch)
        def _():
            _gcn_stage(src, dst, tab, s0, base0 + (b + 2) * bsz, bsz)

        swait(s1)

        @pl.when(b + 3 < nbatch)
        def _():
            _gcn_stage(src, dst, tab, s1, base0 + (b + 3) * bsz, bsz)

    if nbatch % 2:
        gwait(s0)
        sstart(s0)
        swait(s0)


def _sc_l0_body(hv0, ha0, svv, dvv, saa, daa, z2d,
                vout, aout,
                acc, is0, id0, r0, is1, id1, r1, gs0, ss0, gs1, ss1):
    c = lax.axis_index("c")
    t = lax.axis_index("s")
    ept = _E // 16
    nb = ept // _B0
    bufs = ((is0, id0, r0, gs0, ss0), (is1, id1, r1, gs1, ss1))
    _zero_acc(z2d, acc, t)

    @pl.when(c == 0)
    def _():
        _gcn_prime(svv, dvv, hv0, bufs, t * ept, nb, _B0)

    @pl.when(c == 1)
    def _():
        _gcn_prime(saa, daa, ha0, bufs, t * ept, nb, _B0)

    plsc.subcore_barrier()

    @pl.when(c == 0)
    def _():
        _gcn_pipe(svv, dvv, hv0, acc, bufs, t * ept, nb, _B0)

    @pl.when(c == 1)
    def _():
        _gcn_pipe(saa, daa, ha0, acc, bufs, t * ept, nb, _B0)

    plsc.subcore_barrier()

    @pl.when(c == 0)
    def _():
        pltpu.sync_copy(acc.at[pl.ds(t * _RPT, _RPT)],
                        vout.at[pl.ds(t * _RPT, _RPT)])

    @pl.when(c == 1)
    def _():
        pltpu.sync_copy(acc.at[pl.ds(t * _RPT, _RPT)],
                        aout.at[pl.ds(t * _RPT, _RPT)])


def _sc_l0(hv0, ha0, svv, dvv, saa, daa, z2d):
    fn = pl.kernel(
        _sc_l0_body,
        out_type=[jax.ShapeDtypeStruct((_NP, _H), _f32),
                  jax.ShapeDtypeStruct((_NP, _H), _f32)],
        mesh=_MESH,
        scratch_types=[
            pltpu.VMEM_SHARED((_NP, _H), _f32),
            pltpu.VMEM((_B0,), jnp.int32),
            pltpu.VMEM((_B0,), jnp.int32),
            pltpu.VMEM((_B0, _H), _f32),
            pltpu.VMEM((_B0,), jnp.int32),
            pltpu.VMEM((_B0,), jnp.int32),
            pltpu.VMEM((_B0, _H), _f32),
            pltpu.SemaphoreType.DMA,
            pltpu.SemaphoreType.DMA,
            pltpu.SemaphoreType.DMA,
            pltpu.SemaphoreType.DMA,
        ],
    )
    return fn(hv0, ha0, svv, dvv, saa, daa, z2d)


def _sc_l1_body(hv1, ha1, hs, s1, s2, sh16, svv, dvv, saa, daa, sva, dva,
                z2d, z1d,
                v1p, a1p, nump, denp,
                acc, den, is0, id0, r0, is1, id1, r1,
                gisA0, gidA0, eA0, e2A0, ebA0,
                gisA1, gidA1, eA1, e2A1, ebA1,
                shv, dbuf,
                gs0, ss0, gs1, ss1,
                sg10, sg20, sd0, sr0w, sg11, sg21, sd1, sr1w):
    c = lax.axis_index("c")
    t = lax.axis_index("s")
    half = _E // 2
    ept = half // 16
    nb = ept // _B1
    base0 = c * half + t * ept
    out_row0 = c * _NP + t * _RPT
    bufs = ((is0, id0, r0, gs0, ss0), (is1, id1, r1, gs1, ss1))

    pltpu.sync_copy(sh16, shv)

    _zero_acc(z2d, acc, t)
    _gcn_prime(svv, dvv, hv1, bufs, base0, nb, _B1)
    plsc.subcore_barrier()
    _gcn_pipe(svv, dvv, hv1, acc, bufs, base0, nb, _B1)
    plsc.subcore_barrier()
    pltpu.sync_copy(acc.at[pl.ds(t * _RPT, _RPT)],
                    v1p.at[pl.ds(out_row0, _RPT)])

    _zero_acc(z2d, acc, t)
    _gcn_prime(saa, daa, ha1, bufs, base0, nb, _B1)
    plsc.subcore_barrier()
    _gcn_pipe(saa, daa, ha1, acc, bufs, base0, nb, _B1)
    plsc.subcore_barrier()
    pltpu.sync_copy(acc.at[pl.ds(t * _RPT, _RPT)],
                    a1p.at[pl.ds(out_row0, _RPT)])

    _zero_acc(z2d, acc, t)
    pltpu.sync_copy(z1d, dbuf)
    pltpu.sync_copy(dbuf, den.at[pl.ds(t * _RPT, _RPT)])
    shvec = shv[...]
    nbg = ept // _BG
    S0 = (gisA0, gidA0, eA0, e2A0, ebA0, r0, sg10, sg20, sr0w, sd0)
    S1 = (gisA1, gidA1, eA1, e2A1, ebA1, r1, sg11, sg21, sr1w, sd1)

    def gstage(bi, S):
        gis, gid, e1, e2, eb, rws, se1, se2, sr, sd = S
        base = base0 + bi * _BG
        pltpu.sync_copy(sva.at[pl.ds(base, _BG)], gis)
        pltpu.sync_copy(dva.at[pl.ds(base, _BG)], gid)
        pltpu.async_copy(s1.at[gis], e1, se1)
        pltpu.async_copy(s2.at[gid], e2, se2)
        pltpu.async_copy(hs.at[gis], rws, sr)

    def gprocess(S):
        gis, gid, e1, e2, eb, rws, se1, se2, sr, sd = S
        pltpu.make_async_copy(s1.at[gis], e1, se1).wait()
        pltpu.make_async_copy(s2.at[gid], e2, se2).wait()

        @pl.loop(0, _BG // 16)
        def _(i):
            e = e1[pl.ds(i * 16, 16)] + e2[pl.ds(i * 16, 16)]
            e = jnp.maximum(e, 0.2 * e)
            eb[pl.ds(i * 16, 16)] = jnp.exp(e - shvec)

        pltpu.async_copy(eb, den.at[gid], sd, add=True)
        pltpu.make_async_copy(hs.at[gis], rws, sr).wait()

        @pl.loop(0, _BG // 16)
        def _(i16):
            wv = eb[pl.ds(i16 * 16, 16)]
            for l in range(16):
                w = jnp.full((16,), wv[l], _f32)
                rr = i16 * 16 + l
                for j in range(8):
                    rws[rr, pl.ds(j * 16, 16)] = rws[rr, pl.ds(j * 16, 16)] * w

        pltpu.async_copy(rws, acc.at[gid], sr, add=True)

    def gdrain(S):
        gis, gid, e1, e2, eb, rws, se1, se2, sr, sd = S
        pltpu.make_async_copy(eb, den.at[gid], sd).wait()
        pltpu.make_async_copy(rws, acc.at[gid], sr).wait()

    gstage(0, S0)
    gstage(1, S1)
    plsc.subcore_barrier()

    @pl.loop(0, nbg // 2)
    def _(j):
        gb = 2 * j
        gprocess(S0)
        gprocess(S1)
        gdrain(S0)

        @pl.when(gb + 2 < nbg)
        def _():
            gstage(gb + 2, S0)

        gdrain(S1)

        @pl.when(gb + 3 < nbg)
        def _():
            gstage(gb + 3, S1)

    if nbg % 2:
        gprocess(S0)
        gdrain(S0)

    plsc.subcore_barrier()
    pltpu.sync_copy(acc.at[pl.ds(t * _RPT, _RPT)],
                    nump.at[pl.ds(out_row0, _RPT)])

    pltpu.sync_copy(den.at[pl.ds(t * _RPT, _RPT)], dbuf)
    pltpu.sync_copy(dbuf, denp.at[pl.ds(out_row0, _RPT)])


def _sc_l1(hv1, ha1, hs, s1, s2, sh16, svv, dvv, saa, daa, sva, dva, z2d, z1d):
    fn = pl.kernel(
        _sc_l1_body,
        out_type=[jax.ShapeDtypeStruct((2 * _NP, _H), _f32),
                  jax.ShapeDtypeStruct((2 * _NP, _H), _f32),
                  jax.ShapeDtypeStruct((2 * _NP, _H), _f32),
                  jax.ShapeDtypeStruct((2 * _NP,), _f32)],
        mesh=_MESH,
        scratch_types=(
            [pltpu.VMEM_SHARED((_NP, _H), _f32),
             pltpu.VMEM_SHARED((_NP,), _f32),
             pltpu.VMEM((_B1,), jnp.int32),
             pltpu.VMEM((_B1,), jnp.int32),
             pltpu.VMEM((_B1, _H), _f32),
             pltpu.VMEM((_B1,), jnp.int32),
             pltpu.VMEM((_B1,), jnp.int32),
             pltpu.VMEM((_B1, _H), _f32)]
            + 2 * [pltpu.VMEM((_BG,), jnp.int32),
                   pltpu.VMEM((_BG,), jnp.int32),
                   pltpu.VMEM((_BG,), _f32),
                   pltpu.VMEM((_BG,), _f32),
                   pltpu.VMEM((_BG,), _f32)]
            + [pltpu.VMEM((16,), _f32),
               pltpu.VMEM((_RPT,), _f32)]
            + 12 * [pltpu.SemaphoreType.DMA]
        ),
    )
    return fn(hv1, ha1, hs, s1, s2, sh16, svv, dvv, saa, daa, sva, dva,
              z2d, z1d)


def kernel(x_video, x_audio, edge_index_vv, edge_index_aa, edge_index_va,
           W_vv0, b_vv0, W_aa0, b_aa0, W_sh1, b_sh1,
           Wg_src, Wg_dst, a_src, a_dst, b_gat,
           ln0v_g, ln0v_b, ln0a_g, ln0a_b, ln1v_g, ln1v_b, ln1a_g, ln1a_b,
           attw_a, attw_v, lin_W, lin_b):
    svv, dvv = edge_index_vv[0], edge_index_vv[1]
    saa, daa = edge_index_aa[0], edge_index_aa[1]
    sva, dva = edge_index_va[0], edge_index_va[1]
    z2d = jnp.zeros((_RPT, _H), _f32)
    z1d = jnp.zeros((_RPT,), _f32)

    hv0, ha0 = _tc1(x_video, x_audio, W_vv0, W_aa0)
    vraw, araw = _sc_l0(hv0, ha0, svv, dvv, saa, daa, z2d)
    vraw, araw = vraw[:_N], araw[:_N]
    hv1, ha1, hs, s1, s2, shift = _tc2(
        vraw, araw, b_vv0, b_aa0, ln0v_g, ln0v_b, ln0a_g, ln0a_b,
        W_sh1, Wg_src, Wg_dst, a_src, a_dst)
    sh16 = shift[0, :16]
    v1p, a1p, nump, denp = _sc_l1(
        hv1, ha1, hs, s1[:, 0], s2[:, 0], sh16,
        svv, dvv, saa, daa, sva, dva, z2d, z1d)
    v1, a1, sv, sa = _tc3a(v1p, a1p, nump, denp.reshape(2 * _NP, 1),
                           b_sh1, b_gat, ln1v_g, ln1v_b, ln1a_g, ln1a_b,
                           attw_a, attw_v)
    out = _tc3b(v1, a1, sv, sa, lin_W, lin_b)
    return out
